# baseline (speedup 1.0000x reference)
import functools

import jax
import jax.numpy as jnp
from jax.experimental import pallas as pl
from jax.experimental.pallas import tpu as pltpu

_G = 512
_STEPS = 6
_B = 2048
_W = 32
_GPAD = _G + _W

def _split(a):
    hi = a.astype(jnp.bfloat16)
    lo = (a - hi.astype(jnp.float32)).astype(jnp.bfloat16)
    return hi, lo


def _dot3(a_split, b_split, dims):
    a_hi, a_lo = a_split
    b_hi, b_lo = b_split
    d = lambda a, b: jax.lax.dot_general(
        a, b, (dims, ((), ())), preferred_element_type=jnp.float32)
    return d(a_hi, b_hi) + (d(a_hi, b_lo) + d(a_lo, b_hi))


def _dot1(a_hi, b_hi, dims):
    return jax.lax.dot_general(a_hi, b_hi, (dims, ((), ())),
                               preferred_element_type=jnp.float32)


def _accumulate_window(x_hi, seg_row, start, w, h_s, m_s, z_s, racc_s):
    h_win = h_s[pl.ds(start, w), :]
    iota_w = jax.lax.broadcasted_iota(jnp.int32, (w, 1), 0) + start
    sel = iota_w == seg_row
    logits = _dot1(h_win.astype(jnp.bfloat16), x_hi, ((1,), (1,)))
    masked = jnp.where(sel, logits, -jnp.inf)
    m_part = jnp.max(masked, axis=1, keepdims=True)
    m_old = m_s[pl.ds(start, w), :]
    m_new = jnp.maximum(m_old, m_part)
    alpha = jnp.exp(m_old - m_new)
    p = jnp.exp(masked - m_new)
    z_s[pl.ds(start, w), :] = (z_s[pl.ds(start, w), :] * alpha
                               + jnp.sum(p, axis=1, keepdims=True))
    racc_part = _dot1(p.astype(jnp.bfloat16), x_hi, ((1,), (0,)))
    racc_s[pl.ds(start, w), :] = (racc_s[pl.ds(start, w), :] * alpha
                                  + racc_part)
    m_s[pl.ds(start, w), :] = m_new


def _body(xh_ref, seg_ref, bounds_ref, wih_ref, whh_ref, bih_ref, bhh_ref,
          w1_ref, b1_ref, w2_ref, b2_ref, out_ref, h_s, c_s, r_s, racc_s,
          m_s, z_s, *, nb, h_dim):
    s = pl.program_id(0)
    b = pl.program_id(1)

    @pl.when(jnp.logical_and(s == 0, b == 0))
    def _():
        h_s[...] = jnp.zeros_like(h_s)
        c_s[...] = jnp.zeros_like(c_s)

    @pl.when(b == 0)
    def _():
        m_s[...] = jnp.full_like(m_s, -1e30)
        z_s[...] = jnp.zeros_like(z_s)
        racc_s[...] = jnp.zeros_like(racc_s)

    x_hi = xh_ref[...]
    seg_row = seg_ref[0]
    lo = bounds_ref[0, 0, 0]
    hi = bounds_ref[0, 0, 1]
    start = (lo // 8) * 8

    @pl.when(hi - start < _W)
    def _():
        _accumulate_window(x_hi, seg_row, start, _W,
                           h_s, m_s, z_s, racc_s)

    @pl.when(hi - start >= _W)
    def _():
        def chunk(ci, _):
            _accumulate_window(x_hi, seg_row, ci * _W, _W,
                               h_s, m_s, z_s, racc_s)
            return 0
        jax.lax.fori_loop(0, _GPAD // _W, chunk, 0)

    @pl.when(b == nb - 1)
    def _():
        r = racc_s[:_G, :] / (z_s[:_G, :] + 1e-16)
        r_s[...] = r
        h = h_s[:_G, :]
        lstm_in = jnp.concatenate([h, r], axis=1)
        gates = (_dot3(_split(lstm_in), _split(wih_ref[...]), ((1,), (1,)))
                 + bih_ref[...]
                 + _dot3(_split(h), _split(whh_ref[...]), ((1,), (1,)))
                 + bhh_ref[...])
        i_g = jax.nn.sigmoid(gates[:, :h_dim])
        f_g = jax.nn.sigmoid(gates[:, h_dim:2 * h_dim])
        g_g = jnp.tanh(gates[:, 2 * h_dim:3 * h_dim])
        o_g = jax.nn.sigmoid(gates[:, 3 * h_dim:])
        c = f_g * c_s[...] + i_g * g_g
        c_s[...] = c
        h_s[:_G, :] = o_g * jnp.tanh(c)

    @pl.when(jnp.logical_and(b == nb - 1, s == _STEPS - 1))
    def _():
        graph_emb = jnp.concatenate([h_s[:_G, :], r_s[...]], axis=1)
        hidden = jnp.maximum(
            _dot3(_split(graph_emb), _split(w1_ref[...]), ((1,), (1,)))
            + b1_ref[...], 0.0)
        out_ref[...] = _dot3(_split(hidden), _split(w2_ref[...]),
                             ((1,), (1,))) + b2_ref[...]


def kernel(node_embeddings, batch_indices, W_ih, W_hh, b_ih, b_hh,
           W1, b1, W2, b2):
    n, h_dim = node_embeddings.shape
    out_dim = W2.shape[0]
    nb = -(-n // _B)
    npad = nb * _B

    x = jnp.pad(node_embeddings, ((0, npad - n), (0, 0)))
    x_hi = x.astype(jnp.bfloat16)
    seg = jnp.pad(batch_indices.astype(jnp.int32), (0, npad - n),
                  constant_values=_G)
    seg3 = seg.reshape(nb, 1, _B)
    seg2 = seg.reshape(nb, _B)
    bounds = jnp.stack([seg2[:, 0], seg2[:, -1]], axis=1).reshape(nb, 1, 2)

    body = functools.partial(_body, nb=nb, h_dim=h_dim)
    full = lambda shape: pl.BlockSpec(shape, lambda s, b: (0,) * len(shape))

    out = pl.pallas_call(
        body,
        grid=(_STEPS, nb),
        in_specs=[
            pl.BlockSpec((_B, h_dim), lambda s, b: (b, 0)),
            pl.BlockSpec((1, 1, _B), lambda s, b: (b, 0, 0)),
            pl.BlockSpec((1, 1, 2), lambda s, b: (b, 0, 0),
                         memory_space=pltpu.SMEM),
            full(W_ih.shape),
            full(W_hh.shape),
            full((1, b_ih.shape[0])),
            full((1, b_hh.shape[0])),
            full(W1.shape),
            full((1, b1.shape[0])),
            full(W2.shape),
            full((1, b2.shape[0])),
        ],
        out_specs=full((_G, out_dim)),
        out_shape=jax.ShapeDtypeStruct((_G, out_dim), jnp.float32),
        scratch_shapes=[
            pltpu.VMEM((_GPAD, h_dim), jnp.float32),
            pltpu.VMEM((_G, h_dim), jnp.float32),
            pltpu.VMEM((_G, h_dim), jnp.float32),
            pltpu.VMEM((_GPAD, h_dim), jnp.float32),
            pltpu.VMEM((_GPAD, 1), jnp.float32),
            pltpu.VMEM((_GPAD, 1), jnp.float32),
        ],
        compiler_params=pltpu.CompilerParams(
            dimension_semantics=("arbitrary", "arbitrary")),
    )(x_hi, seg3, bounds, W_ih, W_hh, b_ih.reshape(1, -1),
      b_hh.reshape(1, -1),
      W1, b1.reshape(1, -1), W2, b2.reshape(1, -1))
    return out


# speedup vs baseline: 1.0687x; 1.0687x over previous
import functools

import jax
import jax.numpy as jnp
from jax.experimental import pallas as pl
from jax.experimental.pallas import tpu as pltpu

_G = 512
_STEPS = 6
_B = 4096
_W = 64
_GPAD = _G + _W

def _split(a):
    hi = a.astype(jnp.bfloat16)
    lo = (a - hi.astype(jnp.float32)).astype(jnp.bfloat16)
    return hi, lo


def _dot3(a_split, b_split, dims):
    a_hi, a_lo = a_split
    b_hi, b_lo = b_split
    d = lambda a, b: jax.lax.dot_general(
        a, b, (dims, ((), ())), preferred_element_type=jnp.float32)
    return d(a_hi, b_hi) + (d(a_hi, b_lo) + d(a_lo, b_hi))


def _dot1(a_hi, b_hi, dims):
    return jax.lax.dot_general(a_hi, b_hi, (dims, ((), ())),
                               preferred_element_type=jnp.float32)


def _accumulate_window(x_hi, seg_row, start, w, h_s, m_s, z_s, racc_s):
    h_win = h_s[pl.ds(start, w), :]
    iota_w = jax.lax.broadcasted_iota(jnp.int32, (w, 1), 0) + start
    sel = iota_w == seg_row
    logits = _dot1(h_win.astype(jnp.bfloat16), x_hi, ((1,), (1,)))
    masked = jnp.where(sel, logits, -jnp.inf)
    m_part = jnp.max(masked, axis=1, keepdims=True)
    m_old = m_s[pl.ds(start, w), :]
    m_new = jnp.maximum(m_old, m_part)
    alpha = jnp.exp(m_old - m_new)
    p = jnp.exp(masked - m_new)
    z_s[pl.ds(start, w), :] = (z_s[pl.ds(start, w), :] * alpha
                               + jnp.sum(p, axis=1, keepdims=True))
    racc_part = _dot1(p.astype(jnp.bfloat16), x_hi, ((1,), (0,)))
    racc_s[pl.ds(start, w), :] = (racc_s[pl.ds(start, w), :] * alpha
                                  + racc_part)
    m_s[pl.ds(start, w), :] = m_new


def _body(xh_ref, seg_ref, bounds_ref, wih_ref, whh_ref, bih_ref, bhh_ref,
          w1_ref, b1_ref, w2_ref, b2_ref, out_ref, h_s, c_s, r_s, racc_s,
          m_s, z_s, *, nb, h_dim):
    s = pl.program_id(0)
    b = pl.program_id(1)

    @pl.when(jnp.logical_and(s == 0, b == 0))
    def _():
        h_s[...] = jnp.zeros_like(h_s)
        c_s[...] = jnp.zeros_like(c_s)

    @pl.when(b == 0)
    def _():
        m_s[...] = jnp.full_like(m_s, -1e30)
        z_s[...] = jnp.zeros_like(z_s)
        racc_s[...] = jnp.zeros_like(racc_s)

    x_hi = xh_ref[...]
    seg_row = seg_ref[0]
    lo = bounds_ref[0, 0, 0]
    hi = bounds_ref[0, 0, 1]
    start = (lo // 8) * 8

    @pl.when(hi - start < _W)
    def _():
        _accumulate_window(x_hi, seg_row, start, _W,
                           h_s, m_s, z_s, racc_s)

    @pl.when(hi - start >= _W)
    def _():
        def chunk(ci, _):
            _accumulate_window(x_hi, seg_row, ci * _W, _W,
                               h_s, m_s, z_s, racc_s)
            return 0
        jax.lax.fori_loop(0, _GPAD // _W, chunk, 0)

    @pl.when(b == nb - 1)
    def _():
        r = racc_s[:_G, :] / (z_s[:_G, :] + 1e-16)
        r_s[...] = r
        h = h_s[:_G, :]
        lstm_in = jnp.concatenate([h, r], axis=1)
        gates = (_dot3(_split(lstm_in), _split(wih_ref[...]), ((1,), (1,)))
                 + bih_ref[...]
                 + _dot3(_split(h), _split(whh_ref[...]), ((1,), (1,)))
                 + bhh_ref[...])
        i_g = jax.nn.sigmoid(gates[:, :h_dim])
        f_g = jax.nn.sigmoid(gates[:, h_dim:2 * h_dim])
        g_g = jnp.tanh(gates[:, 2 * h_dim:3 * h_dim])
        o_g = jax.nn.sigmoid(gates[:, 3 * h_dim:])
        c = f_g * c_s[...] + i_g * g_g
        c_s[...] = c
        h_s[:_G, :] = o_g * jnp.tanh(c)

    @pl.when(jnp.logical_and(b == nb - 1, s == _STEPS - 1))
    def _():
        graph_emb = jnp.concatenate([h_s[:_G, :], r_s[...]], axis=1)
        hidden = jnp.maximum(
            _dot3(_split(graph_emb), _split(w1_ref[...]), ((1,), (1,)))
            + b1_ref[...], 0.0)
        out_ref[...] = _dot3(_split(hidden), _split(w2_ref[...]),
                             ((1,), (1,))) + b2_ref[...]


def kernel(node_embeddings, batch_indices, W_ih, W_hh, b_ih, b_hh,
           W1, b1, W2, b2):
    n, h_dim = node_embeddings.shape
    out_dim = W2.shape[0]
    nb = -(-n // _B)
    npad = nb * _B

    x = jnp.pad(node_embeddings, ((0, npad - n), (0, 0)))
    x_hi = x.astype(jnp.bfloat16)
    seg = jnp.pad(batch_indices.astype(jnp.int32), (0, npad - n),
                  constant_values=_G)
    seg3 = seg.reshape(nb, 1, _B)
    seg2 = seg.reshape(nb, _B)
    bounds = jnp.stack([seg2[:, 0], seg2[:, -1]], axis=1).reshape(nb, 1, 2)

    body = functools.partial(_body, nb=nb, h_dim=h_dim)
    full = lambda shape: pl.BlockSpec(shape, lambda s, b: (0,) * len(shape))

    out = pl.pallas_call(
        body,
        grid=(_STEPS, nb),
        in_specs=[
            pl.BlockSpec((_B, h_dim), lambda s, b: (b, 0)),
            pl.BlockSpec((1, 1, _B), lambda s, b: (b, 0, 0)),
            pl.BlockSpec((1, 1, 2), lambda s, b: (b, 0, 0),
                         memory_space=pltpu.SMEM),
            full(W_ih.shape),
            full(W_hh.shape),
            full((1, b_ih.shape[0])),
            full((1, b_hh.shape[0])),
            full(W1.shape),
            full((1, b1.shape[0])),
            full(W2.shape),
            full((1, b2.shape[0])),
        ],
        out_specs=full((_G, out_dim)),
        out_shape=jax.ShapeDtypeStruct((_G, out_dim), jnp.float32),
        scratch_shapes=[
            pltpu.VMEM((_GPAD, h_dim), jnp.float32),
            pltpu.VMEM((_G, h_dim), jnp.float32),
            pltpu.VMEM((_G, h_dim), jnp.float32),
            pltpu.VMEM((_GPAD, h_dim), jnp.float32),
            pltpu.VMEM((_GPAD, 1), jnp.float32),
            pltpu.VMEM((_GPAD, 1), jnp.float32),
        ],
        compiler_params=pltpu.CompilerParams(
            dimension_semantics=("arbitrary", "arbitrary")),
    )(x_hi, seg3, bounds, W_ih, W_hh, b_ih.reshape(1, -1),
      b_hh.reshape(1, -1),
      W1, b1.reshape(1, -1), W2, b2.reshape(1, -1))
    return out


# speedup vs baseline: 1.1252x; 1.0528x over previous
import functools

import jax
import jax.numpy as jnp
from jax.experimental import pallas as pl
from jax.experimental.pallas import tpu as pltpu

_G = 512
_STEPS = 6
_B = 4096
_W = 64
_GPAD = _G + _W


def _split(a):
    hi = a.astype(jnp.bfloat16)
    lo = (a - hi.astype(jnp.float32)).astype(jnp.bfloat16)
    return hi, lo


def _dot3(a_split, b_split, dims):
    a_hi, a_lo = a_split
    b_hi, b_lo = b_split
    d = lambda a, b: jax.lax.dot_general(
        a, b, (dims, ((), ())), preferred_element_type=jnp.float32)
    return d(a_hi, b_hi) + (d(a_hi, b_lo) + d(a_lo, b_hi))


def _dot1(a_hi, b_hi, dims):
    return jax.lax.dot_general(a_hi, b_hi, (dims, ((), ())),
                               preferred_element_type=jnp.float32)


def _window_logits(x_hi, start, w, h_s):
    h_win = h_s[pl.ds(start, w), :]
    return _dot1(h_win.astype(jnp.bfloat16), x_hi, ((1,), (1,)))


def _softmax_scatter(logits, x_hi, seg_row, start, w, m_s, z_s, racc_s):
    iota_w = jax.lax.broadcasted_iota(jnp.int32, (w, 1), 0) + start
    sel = iota_w == seg_row
    masked = jnp.where(sel, logits, -jnp.inf)
    m_part = jnp.max(masked, axis=1, keepdims=True)
    m_old = m_s[pl.ds(start, w), :]
    m_new = jnp.maximum(m_old, m_part)
    alpha = jnp.exp(m_old - m_new)
    p = jnp.exp(masked - m_new)
    z_s[pl.ds(start, w), :] = (z_s[pl.ds(start, w), :] * alpha
                               + jnp.sum(p, axis=1, keepdims=True))
    racc_part = _dot1(p.astype(jnp.bfloat16), x_hi, ((1,), (0,)))
    racc_s[pl.ds(start, w), :] = (racc_s[pl.ds(start, w), :] * alpha
                                  + racc_part)
    m_s[pl.ds(start, w), :] = m_new


def _body(xc_ref, xp_ref, segp_ref, bc_ref, bp_ref, wih_ref, whh_ref,
          bih_ref, bhh_ref, w1_ref, b1_ref, w2_ref, b2_ref, out_ref,
          h_s, c_s, r_s, racc_s, m_s, z_s, lg_s, *, nb, h_dim):
    s = pl.program_id(0)
    j = pl.program_id(1)

    @pl.when(jnp.logical_and(s == 0, j == 0))
    def _():
        h_s[...] = jnp.zeros_like(h_s)
        c_s[...] = jnp.zeros_like(c_s)

    @pl.when(j == 0)
    def _():
        m_s[...] = jnp.full_like(m_s, -1e30)
        z_s[...] = jnp.zeros_like(z_s)
        racc_s[...] = jnp.zeros_like(racc_s)

    @pl.when(j >= 1)
    def _():
        x_prev = xp_ref[...]
        seg_row = segp_ref[0]
        lo = bp_ref[0, 0, 0]
        hi = bp_ref[0, 0, 1]
        start = (lo // 8) * 8

        @pl.when(hi - start < _W)
        def _():
            _softmax_scatter(lg_s[(j - 1) % 2], x_prev, seg_row, start,
                             _W, m_s, z_s, racc_s)

        @pl.when(hi - start >= _W)
        def _():
            def chunk(ci, _):
                lg = _window_logits(x_prev, ci * _W, _W, h_s)
                _softmax_scatter(lg, x_prev, seg_row, ci * _W, _W,
                                 m_s, z_s, racc_s)
                return 0
            jax.lax.fori_loop(0, _GPAD // _W, chunk, 0)

    @pl.when(j <= nb - 1)
    def _():
        lo = bc_ref[0, 0, 0]
        hi = bc_ref[0, 0, 1]
        start = (lo // 8) * 8

        @pl.when(hi - start < _W)
        def _():
            lg_s[j % 2] = _window_logits(xc_ref[...], start, _W, h_s)

    @pl.when(j == nb)
    def _():
        r = racc_s[:_G, :] / (z_s[:_G, :] + 1e-16)
        r_s[...] = r
        h = h_s[:_G, :]
        lstm_in = jnp.concatenate([h, r], axis=1)
        gates = (_dot3(_split(lstm_in), _split(wih_ref[...]), ((1,), (1,)))
                 + bih_ref[...]
                 + _dot3(_split(h), _split(whh_ref[...]), ((1,), (1,)))
                 + bhh_ref[...])
        i_g = jax.nn.sigmoid(gates[:, :h_dim])
        f_g = jax.nn.sigmoid(gates[:, h_dim:2 * h_dim])
        g_g = jnp.tanh(gates[:, 2 * h_dim:3 * h_dim])
        o_g = jax.nn.sigmoid(gates[:, 3 * h_dim:])
        c = f_g * c_s[...] + i_g * g_g
        c_s[...] = c
        h_s[:_G, :] = o_g * jnp.tanh(c)

    @pl.when(jnp.logical_and(j == nb, s == _STEPS - 1))
    def _():
        graph_emb = jnp.concatenate([h_s[:_G, :], r_s[...]], axis=1)
        hidden = jnp.maximum(
            _dot3(_split(graph_emb), _split(w1_ref[...]), ((1,), (1,)))
            + b1_ref[...], 0.0)
        out_ref[...] = _dot3(_split(hidden), _split(w2_ref[...]),
                             ((1,), (1,))) + b2_ref[...]


def kernel(node_embeddings, batch_indices, W_ih, W_hh, b_ih, b_hh,
           W1, b1, W2, b2):
    n, h_dim = node_embeddings.shape
    out_dim = W2.shape[0]
    nb = -(-n // _B)
    npad = nb * _B

    x = jnp.pad(node_embeddings, ((0, npad - n), (0, 0)))
    x_hi = x.astype(jnp.bfloat16)
    seg = jnp.pad(batch_indices.astype(jnp.int32), (0, npad - n),
                  constant_values=_G)
    seg3 = seg.reshape(nb, 1, _B)
    seg2 = seg.reshape(nb, _B)
    bounds = jnp.stack([seg2[:, 0], seg2[:, -1]], axis=1).reshape(nb, 1, 2)

    body = functools.partial(_body, nb=nb, h_dim=h_dim)
    full = lambda shape: pl.BlockSpec(shape, lambda s, j: (0,) * len(shape))
    cur = lambda s, j: (jnp.minimum(j, nb - 1), 0)
    prev = lambda s, j: (jnp.maximum(j - 1, 0), 0)

    out = pl.pallas_call(
        body,
        grid=(_STEPS, nb + 1),
        in_specs=[
            pl.BlockSpec((_B, h_dim), cur),
            pl.BlockSpec((_B, h_dim), prev),
            pl.BlockSpec((1, 1, _B),
                         lambda s, j: (jnp.maximum(j - 1, 0), 0, 0)),
            pl.BlockSpec((1, 1, 2),
                         lambda s, j: (jnp.minimum(j, nb - 1), 0, 0),
                         memory_space=pltpu.SMEM),
            pl.BlockSpec((1, 1, 2),
                         lambda s, j: (jnp.maximum(j - 1, 0), 0, 0),
                         memory_space=pltpu.SMEM),
            full(W_ih.shape),
            full(W_hh.shape),
            full((1, b_ih.shape[0])),
            full((1, b_hh.shape[0])),
            full(W1.shape),
            full((1, b1.shape[0])),
            full(W2.shape),
            full((1, b2.shape[0])),
        ],
        out_specs=full((_G, out_dim)),
        out_shape=jax.ShapeDtypeStruct((_G, out_dim), jnp.float32),
        scratch_shapes=[
            pltpu.VMEM((_GPAD, h_dim), jnp.float32),
            pltpu.VMEM((_G, h_dim), jnp.float32),
            pltpu.VMEM((_G, h_dim), jnp.float32),
            pltpu.VMEM((_GPAD, h_dim), jnp.float32),
            pltpu.VMEM((_GPAD, 1), jnp.float32),
            pltpu.VMEM((_GPAD, 1), jnp.float32),
            pltpu.VMEM((2, _W, _B), jnp.float32),
        ],
        compiler_params=pltpu.CompilerParams(
            dimension_semantics=("arbitrary", "arbitrary")),
    )(x_hi, x_hi, seg3, bounds, bounds, W_ih, W_hh, b_ih.reshape(1, -1),
      b_hh.reshape(1, -1), W1, b1.reshape(1, -1), W2, b2.reshape(1, -1))
    return out
